# Initial kernel scaffold; baseline (speedup 1.0000x reference)
import jax
import jax.numpy as jnp
from jax.experimental import pallas as pl


def kernel(X, E, X_q, E_q, edge_index_g, graph_id_g, edge_index_q, graph_id_q, W_g_emb, b_g_emb, W_q_emb, b_q_emb, beta_g, beta_q, red_W1, red_b1, red_W2, red_b2, Wp1, bp1, Wp2, bp2):
    raise NotImplementedError("write your pallas kernel here")


# speedup vs baseline: 1.9227x; 1.9227x over previous
import functools

import jax
import jax.numpy as jnp
from jax.experimental import pallas as pl
from jax.experimental.pallas import tpu as pltpu

L = 2
NUM_GRAPHS = 16


def _mm_bias_body(x_ref, w_ref, b_ref, o_ref):
    o_ref[...] = (
        jnp.dot(x_ref[...], w_ref[...], preferred_element_type=jnp.float32)
        + b_ref[...]
    )


def _mm_bias(x, w, b, block_rows):
    n, k = x.shape
    m = w.shape[1]
    grid = n // block_rows
    return pl.pallas_call(
        _mm_bias_body,
        grid=(grid,),
        in_specs=[
            pl.BlockSpec((block_rows, k), lambda i: (i, 0)),
            pl.BlockSpec((k, m), lambda i: (0, 0)),
            pl.BlockSpec((m,), lambda i: (0,)),
        ],
        out_specs=pl.BlockSpec((block_rows, m), lambda i: (i, 0)),
        out_shape=jax.ShapeDtypeStruct((n, m), jnp.float32),
    )(x, w, b)


def _agnn(h, src, dst, beta, n):
    hn = h / (jnp.linalg.norm(h, axis=1, keepdims=True) + 1e-12)
    e = beta * jnp.sum(hn[src] * hn[dst], axis=1)
    ex = jnp.exp(e)
    den = jax.ops.segment_sum(ex, dst, num_segments=n)
    num = jax.ops.segment_sum(ex[:, None] * h[src], dst, num_segments=n)
    return num / (den + 1e-12)[:, None]


def kernel(X, E, X_q, E_q, edge_index_g, graph_id_g, edge_index_q, graph_id_q,
           W_g_emb, b_g_emb, W_q_emb, b_q_emb, beta_g, beta_q,
           red_W1, red_b1, red_W2, red_b2, Wp1, bp1, Wp2, bp2):
    n_g = X.shape[0]
    n_q = X_q.shape[0]
    src_g, dst_g = edge_index_g[0], edge_index_g[1]
    src_q, dst_q = edge_index_q[0], edge_index_q[1]
    h_g = _mm_bias(X, W_g_emb, b_g_emb, 1000)
    h_q = _mm_bias(X_q, W_q_emb, b_q_emb, 1000)
    for l in range(L):
        h_q = _agnn(h_q, src_q, dst_q, beta_q[l], n_q)
        h_q_aggr = jax.ops.segment_sum(h_q, graph_id_q, num_segments=NUM_GRAPHS)
        h_cat = jnp.concatenate([h_g, h_q_aggr[graph_id_g]], axis=1)
        h_g = _agnn(h_cat, src_g, dst_g, beta_g[l], n_g)
        h_g = jax.nn.relu(h_g @ red_W1[l] + red_b1[l]) @ red_W2[l] + red_b2[l]
    hg_sum = jax.ops.segment_sum(h_g, graph_id_g, num_segments=NUM_GRAPHS)
    hq_sum = jax.ops.segment_sum(h_q, graph_id_q, num_segments=NUM_GRAPHS)
    y = jax.nn.relu(jnp.concatenate([hg_sum, hq_sum], axis=1) @ Wp1 + bp1) @ Wp2 + bp2
    return y


# speedup vs baseline: 2.9219x; 1.5197x over previous
import functools

import jax
import jax.numpy as jnp
from jax import lax
from jax.experimental import pallas as pl
from jax.experimental.pallas import tpu as pltpu
from jax.experimental.pallas import tpu_sc as plsc

L = 2
NUM_GRAPHS = 16
HALF = 64
TW = 128


def _make_edge_dot(n_nodes, n_edges, chunk):
    d_feat = 128
    per_tile = n_edges // 32
    n_chunks = per_tile // chunk
    groups = -(-chunk // 16)
    cpad = groups * 16
    mesh = plsc.VectorSubcoreMesh(core_axis_name="c", subcore_axis_name="s")

    @functools.partial(
        pl.kernel, mesh=mesh,
        out_type=jax.ShapeDtypeStruct((n_edges,), jnp.float32),
        compiler_params=pltpu.CompilerParams(needs_layout_passes=False),
        scratch_types=[
            pltpu.VMEM((cpad,), jnp.int32),
            pltpu.VMEM((cpad,), jnp.int32),
            pltpu.VMEM((cpad, d_feat), jnp.float32),
            pltpu.VMEM((cpad, d_feat), jnp.float32),
            pltpu.VMEM((cpad,), jnp.float32),
            pltpu.VMEM((16,), jnp.float32),
            pltpu.VMEM((n_nodes,), jnp.int32),
            pltpu.VMEM((n_nodes,), jnp.float32),
            pltpu.VMEM((NUM_GRAPHS * NUM_GRAPHS,), jnp.float32),
            pltpu.SemaphoreType.DMA,
            pltpu.SemaphoreType.DMA,
        ],
    )
    def k(hn_hbm, rinv_hbm, gid_hbm, gram_hbm, src_hbm, dst_hbm, beta_hbm,
          w_hbm, si, di, S, T, wb, bb, gid_v, rinv_v, gram_v, sem1, sem2):
        cid = lax.axis_index("c")
        sid = lax.axis_index("s")
        wid = sid * 2 + cid
        base0 = wid * per_tile
        pltpu.sync_copy(beta_hbm, bb)
        pltpu.sync_copy(gid_hbm, gid_v)
        pltpu.sync_copy(rinv_hbm, rinv_v)
        pltpu.sync_copy(gram_hbm, gram_v)
        bv = bb[...]
        lanes = lax.iota(jnp.int32, 16)
        if cpad != chunk:
            si[pl.ds(cpad - 16, 16)] = jnp.zeros((16,), jnp.int32)
            di[pl.ds(cpad - 16, 16)] = jnp.zeros((16,), jnp.int32)

        def chunk_body(i, carry):
            base = base0 + i * chunk
            pltpu.sync_copy(src_hbm.at[pl.ds(base, chunk)],
                            si.at[pl.ds(0, chunk)])
            pltpu.sync_copy(dst_hbm.at[pl.ds(base, chunk)],
                            di.at[pl.ds(0, chunk)])
            cp1 = pltpu.async_copy(hn_hbm.at[si], S, sem1)
            cp2 = pltpu.async_copy(hn_hbm.at[di], T, sem2)
            cp1.wait()
            cp2.wait()
            for g in range(groups):
                rows = g * 16 + lanes
                s16 = si[pl.ds(g * 16, 16)]
                d16 = di[pl.ds(g * 16, 16)]
                gs = plsc.load_gather(gid_v, [s16])
                gd = plsc.load_gather(gid_v, [d16])
                rs = plsc.load_gather(rinv_v, [s16])
                rd = plsc.load_gather(rinv_v, [d16])
                gv = plsc.load_gather(gram_v, [gs * NUM_GRAPHS + gd])

                def dot_body(j, acc):
                    col0 = j * 8
                    for kk in range(8):
                        cv = jnp.full((16,), col0 + kk, jnp.int32)
                        a = plsc.load_gather(S, [rows, cv])
                        b = plsc.load_gather(T, [rows, cv])
                        acc = acc + a * b
                    return acc

                acc = lax.fori_loop(0, d_feat // 8, dot_body,
                                    jnp.zeros((16,), jnp.float32))
                e = acc + gv * rs * rd
                wb[pl.ds(g * 16, 16)] = jnp.exp(e * bv)
            pltpu.sync_copy(wb.at[pl.ds(0, chunk)],
                            w_hbm.at[pl.ds(base, chunk)])
            return carry

        lax.fori_loop(0, n_chunks, chunk_body, 0)

    return k


def _make_aggregate(n_nodes, n_edges, chunk):
    per_tile = n_edges // 16
    n_chunks = per_tile // chunk
    stripe = -(-(n_nodes // 16) // 8) * 8
    last_rows = n_nodes - 15 * stripe
    assert last_rows > 0 and last_rows % 8 == 0
    mrows = 120
    full_a, tail_a = divmod(stripe, mrows)
    full_b, tail_b = divmod(last_rows, mrows)
    kgroups = TW // 16
    mesh = plsc.VectorSubcoreMesh(core_axis_name="c", subcore_axis_name="s")
    osd = jax.ShapeDtypeStruct((n_nodes, TW), jnp.float32)

    @functools.partial(
        pl.kernel, mesh=mesh,
        out_type=(osd, osd),
        compiler_params=pltpu.CompilerParams(needs_layout_passes=False),
        scratch_types=[
            pltpu.VMEM((chunk,), jnp.int32),
            pltpu.VMEM((chunk,), jnp.int32),
            pltpu.VMEM((chunk,), jnp.float32),
            pltpu.VMEM((chunk, TW), jnp.float32),
            pltpu.VMEM((mrows, TW), jnp.float32),
            pltpu.VMEM_SHARED((n_nodes, TW), jnp.float32),
            pltpu.SemaphoreType.DMA,
        ],
    )
    def k(t0_hbm, t1_hbm, src_hbm, dst_hbm, w_hbm, o0_hbm, o1_hbm,
          si, di, wv, R, Z, acc, sem):
        cid = lax.axis_index("c")
        sid = lax.axis_index("s")

        def zrow(r, carry):
            for kk in range(kgroups):
                Z[r, pl.ds(kk * 16, 16)] = jnp.zeros((16,), jnp.float32)
            return carry

        lax.fori_loop(0, mrows, zrow, 0)

        def stripe_copy(fn):
            base = sid * stripe

            def full_chunks(nfull, tail):
                def cb(m, carry):
                    fn(base + m * mrows, mrows)
                    return carry
                lax.fori_loop(0, nfull, cb, 0)
                if tail:
                    fn(base + nfull * mrows, tail)

            if (full_a, tail_a) == (full_b, tail_b):
                full_chunks(full_a, tail_a)
            else:
                pl.when(sid != 15)(lambda: full_chunks(full_a, tail_a))
                pl.when(sid == 15)(lambda: full_chunks(full_b, tail_b))

        stripe_copy(lambda r0, nr: pltpu.sync_copy(
            Z.at[pl.ds(0, nr)], acc.at[pl.ds(r0, nr)]))
        plsc.subcore_barrier()

        def run(t_hbm):
            base0 = sid * per_tile

            def cb(i, carry):
                base = base0 + i * chunk
                pltpu.sync_copy(src_hbm.at[pl.ds(base, chunk)], si)
                pltpu.sync_copy(dst_hbm.at[pl.ds(base, chunk)], di)
                pltpu.sync_copy(w_hbm.at[pl.ds(base, chunk)], wv)
                pltpu.async_copy(t_hbm.at[si], R, sem).wait()

                def erow(e, carry2):
                    ws = plsc.load_gather(
                        wv, [jnp.full((16,), e, jnp.int32)])
                    for kk in range(kgroups):
                        R[e, pl.ds(kk * 16, 16)] = (
                            R[e, pl.ds(kk * 16, 16)] * ws)
                    return carry2

                lax.fori_loop(0, chunk, erow, 0)
                pltpu.sync_copy(R, acc.at[di], add=True)
                return carry

            lax.fori_loop(0, n_chunks, cb, 0)

        pl.when(cid == 0)(lambda: run(t0_hbm))
        pl.when(cid == 1)(lambda: run(t1_hbm))
        plsc.subcore_barrier()

        def dump(o_hbm):
            def one(r0, nr):
                pltpu.sync_copy(acc.at[pl.ds(r0, nr)], Z.at[pl.ds(0, nr)])
                pltpu.sync_copy(Z.at[pl.ds(0, nr)], o_hbm.at[pl.ds(r0, nr)])
            stripe_copy(one)

        pl.when(cid == 0)(lambda: dump(o0_hbm))
        pl.when(cid == 1)(lambda: dump(o1_hbm))

    return k


def _mm_bias_body(x_ref, w_ref, b_ref, o_ref):
    o_ref[...] = (
        jnp.dot(x_ref[...], w_ref[...], preferred_element_type=jnp.float32)
        + b_ref[...]
    )


def _mm_bias(x, w, b, block_rows):
    n, kdim = x.shape
    m = w.shape[1]
    return pl.pallas_call(
        _mm_bias_body,
        grid=(n // block_rows,),
        in_specs=[
            pl.BlockSpec((block_rows, kdim), lambda i: (i, 0)),
            pl.BlockSpec((kdim, m), lambda i: (0, 0)),
            pl.BlockSpec((m,), lambda i: (0,)),
        ],
        out_specs=pl.BlockSpec((block_rows, m), lambda i: (i, 0)),
        out_shape=jax.ShapeDtypeStruct((n, m), jnp.float32),
    )(x, w, b)


def _onehot(gid):
    return (gid[:, None] == lax.iota(jnp.int32, NUM_GRAPHS)[None, :]
            ).astype(jnp.float32)


def _tables(h, oh, rinv):
    r = h.shape[0]
    hn = h * rinv[:, None]
    zer = jnp.zeros((r, TW - HALF - NUM_GRAPHS), jnp.float32)
    t0 = jnp.concatenate([h[:, :HALF], oh, zer], axis=1)
    t1 = jnp.concatenate([h[:, HALF:], oh, zer], axis=1)
    return hn, t0, t1


def _prep_q_body(h_ref, gid_ref, hn_ref, rinv_ref, t0_ref, t1_ref):
    h = h_ref[...]
    rinv = 1.0 / (jnp.sqrt(jnp.sum(h * h, axis=1)) + 1e-12)
    oh = _onehot(gid_ref[0, 0, :])
    hn, t0, t1 = _tables(h, oh, rinv)
    hn_ref[...] = hn
    rinv_ref[...] = rinv[:, None]
    t0_ref[...] = t0
    t1_ref[...] = t1


def _prep_q(h, gid3, block_rows):
    n, d = h.shape
    return pl.pallas_call(
        _prep_q_body,
        grid=(n // block_rows,),
        in_specs=[
            pl.BlockSpec((block_rows, d), lambda i: (i, 0)),
            pl.BlockSpec((1, 1, block_rows), lambda i: (i, 0, 0)),
        ],
        out_specs=[
            pl.BlockSpec((block_rows, d), lambda i: (i, 0)),
            pl.BlockSpec((block_rows, 1), lambda i: (i, 0)),
            pl.BlockSpec((block_rows, TW), lambda i: (i, 0)),
            pl.BlockSpec((block_rows, TW), lambda i: (i, 0)),
        ],
        out_shape=[
            jax.ShapeDtypeStruct((n, d), jnp.float32),
            jax.ShapeDtypeStruct((n, 1), jnp.float32),
            jax.ShapeDtypeStruct((n, TW), jnp.float32),
            jax.ShapeDtypeStruct((n, TW), jnp.float32),
        ],
    )(h, gid3)


def _prep_g_body(h_ref, aggr_ref, gid_ref, hn_ref, rinv_ref, t0_ref,
                 t1_ref, gram_ref):
    h = h_ref[...]
    aggr = aggr_ref[...]
    oh = _onehot(gid_ref[0, 0, :])
    nag = jnp.sum(aggr * aggr, axis=1)
    n2 = jnp.sum(h * h, axis=1) + jnp.dot(
        oh, nag[:, None], preferred_element_type=jnp.float32)[:, 0]
    rinv = 1.0 / (jnp.sqrt(n2) + 1e-12)
    hn, t0, t1 = _tables(h, oh, rinv)
    hn_ref[...] = hn
    rinv_ref[...] = rinv[:, None]
    t0_ref[...] = t0
    t1_ref[...] = t1
    gram_ref[...] = lax.dot_general(
        aggr, aggr, (((1,), (1,)), ((), ())),
        preferred_element_type=jnp.float32)


def _prep_g(h, aggr, gid3, block_rows):
    n, d = h.shape
    return pl.pallas_call(
        _prep_g_body,
        grid=(n // block_rows,),
        in_specs=[
            pl.BlockSpec((block_rows, d), lambda i: (i, 0)),
            pl.BlockSpec((NUM_GRAPHS, d), lambda i: (0, 0)),
            pl.BlockSpec((1, 1, block_rows), lambda i: (i, 0, 0)),
        ],
        out_specs=[
            pl.BlockSpec((block_rows, d), lambda i: (i, 0)),
            pl.BlockSpec((block_rows, 1), lambda i: (i, 0)),
            pl.BlockSpec((block_rows, TW), lambda i: (i, 0)),
            pl.BlockSpec((block_rows, TW), lambda i: (i, 0)),
            pl.BlockSpec((NUM_GRAPHS, NUM_GRAPHS), lambda i: (0, 0)),
        ],
        out_shape=[
            jax.ShapeDtypeStruct((n, d), jnp.float32),
            jax.ShapeDtypeStruct((n, 1), jnp.float32),
            jax.ShapeDtypeStruct((n, TW), jnp.float32),
            jax.ShapeDtypeStruct((n, TW), jnp.float32),
            jax.ShapeDtypeStruct((NUM_GRAPHS, NUM_GRAPHS), jnp.float32),
        ],
    )(h, aggr, gid3)


def _post_q_body(o0_ref, o1_ref, gid_ref, hq_ref, aggr_ref):
    i = pl.program_id(0)
    o0 = o0_ref[...]
    o1 = o1_ref[...]
    den = jnp.sum(o0[:, HALF:HALF + NUM_GRAPHS], axis=1)
    r = (1.0 / (den + 1e-12))[:, None]
    hq = jnp.concatenate([o0[:, :HALF] * r, o1[:, :HALF] * r], axis=1)
    hq_ref[...] = hq
    oh = _onehot(gid_ref[0, 0, :])
    part = lax.dot_general(oh, hq, (((0,), (0,)), ((), ())),
                           preferred_element_type=jnp.float32)

    @pl.when(i == 0)
    def _():
        aggr_ref[...] = jnp.zeros_like(aggr_ref)

    aggr_ref[...] += part


def _post_q(o0, o1, gid3, d, block_rows):
    n = o0.shape[0]
    return pl.pallas_call(
        _post_q_body,
        grid=(n // block_rows,),
        in_specs=[
            pl.BlockSpec((block_rows, TW), lambda i: (i, 0)),
            pl.BlockSpec((block_rows, TW), lambda i: (i, 0)),
            pl.BlockSpec((1, 1, block_rows), lambda i: (i, 0, 0)),
        ],
        out_specs=[
            pl.BlockSpec((block_rows, d), lambda i: (i, 0)),
            pl.BlockSpec((NUM_GRAPHS, d), lambda i: (0, 0)),
        ],
        out_shape=[
            jax.ShapeDtypeStruct((n, d), jnp.float32),
            jax.ShapeDtypeStruct((NUM_GRAPHS, d), jnp.float32),
        ],
    )(o0, o1, gid3)


def _post_g_body(o0_ref, o1_ref, aggr_ref, w1_ref, b1_ref, w2_ref, b2_ref,
                 hg_ref):
    o0 = o0_ref[...]
    o1 = o1_ref[...]
    cnt = o0[:, HALF:HALF + NUM_GRAPHS]
    den = jnp.sum(cnt, axis=1)
    r = (1.0 / (den + 1e-12))[:, None]
    a = o0[:, :HALF] * r
    b = o1[:, :HALF] * r
    c = jnp.dot(cnt, aggr_ref[...],
                preferred_element_type=jnp.float32) * r
    w1 = w1_ref[...]
    t = jax.nn.relu(
        jnp.dot(a, w1[:HALF], preferred_element_type=jnp.float32)
        + jnp.dot(b, w1[HALF:2 * HALF], preferred_element_type=jnp.float32)
        + jnp.dot(c, w1[2 * HALF:], preferred_element_type=jnp.float32)
        + b1_ref[...])
    hg_ref[...] = (
        jnp.dot(t, w2_ref[...], preferred_element_type=jnp.float32)
        + b2_ref[...])


def _post_g(o0, o1, aggr, w1, b1, w2, b2, d, block_rows):
    n = o0.shape[0]
    return pl.pallas_call(
        _post_g_body,
        grid=(n // block_rows,),
        in_specs=[
            pl.BlockSpec((block_rows, TW), lambda i: (i, 0)),
            pl.BlockSpec((block_rows, TW), lambda i: (i, 0)),
            pl.BlockSpec((NUM_GRAPHS, d), lambda i: (0, 0)),
            pl.BlockSpec((2 * d, d), lambda i: (0, 0)),
            pl.BlockSpec((d,), lambda i: (0,)),
            pl.BlockSpec((d, d), lambda i: (0, 0)),
            pl.BlockSpec((d,), lambda i: (0,)),
        ],
        out_specs=pl.BlockSpec((block_rows, d), lambda i: (i, 0)),
        out_shape=jax.ShapeDtypeStruct((n, d), jnp.float32),
    )(o0, o1, aggr, w1, b1, w2, b2)


def _segsum_body(h_ref, gid_ref, out_ref):
    i = pl.program_id(0)
    oh = _onehot(gid_ref[0, 0, :])
    part = lax.dot_general(oh, h_ref[...], (((0,), (0,)), ((), ())),
                           preferred_element_type=jnp.float32)

    @pl.when(i == 0)
    def _():
        out_ref[...] = jnp.zeros_like(out_ref)

    out_ref[...] += part


def _segsum(h, gid3, block_rows):
    n, d = h.shape
    return pl.pallas_call(
        _segsum_body,
        grid=(n // block_rows,),
        in_specs=[
            pl.BlockSpec((block_rows, d), lambda i: (i, 0)),
            pl.BlockSpec((1, 1, block_rows), lambda i: (i, 0, 0)),
        ],
        out_specs=pl.BlockSpec((NUM_GRAPHS, d), lambda i: (0, 0)),
        out_shape=jax.ShapeDtypeStruct((NUM_GRAPHS, d), jnp.float32),
    )(h, gid3)


def _head_body(a_ref, b_ref, w1_ref, b1_ref, w2_ref, b2_ref, y_ref):
    x = jnp.concatenate([a_ref[...], b_ref[...]], axis=1)
    t = jax.nn.relu(
        jnp.dot(x, w1_ref[...], preferred_element_type=jnp.float32)
        + b1_ref[...])
    y_ref[...] = (
        jnp.dot(t, w2_ref[...], preferred_element_type=jnp.float32)
        + b2_ref[...])


def _head(a, b, w1, b1, w2, b2):
    out_d = w2.shape[1]
    return pl.pallas_call(
        _head_body,
        out_shape=jax.ShapeDtypeStruct((NUM_GRAPHS, out_d), jnp.float32),
    )(a, b, w1, b1, w2, b2)


_DBG_XLA_DOT = False
_DBG_XLA_AGGR = False
_DIAG = None


def _xla_edge_dot(hn, rinv, gid, gram, src, dst, b16):
    e = jnp.sum(hn[src] * hn[dst], axis=1)
    g = gram[gid[src] * NUM_GRAPHS + gid[dst]] * rinv[src] * rinv[dst]
    return jnp.exp((e + g) * b16[0])


def _xla_aggr(t0, t1, src, dst, w, n):
    o0 = jax.ops.segment_sum(w[:, None] * t0[src], dst, num_segments=n)
    o1 = jax.ops.segment_sum(w[:, None] * t1[src], dst, num_segments=n)
    return o0, o1

_edge_dot_g = _make_edge_dot(10000, 320000, 80)
_edge_dot_q = _make_edge_dot(2000, 32000, 40)
_aggr_g = _make_aggregate(10000, 320000, 80)
_aggr_q = _make_aggregate(2000, 32000, 80)


def kernel(X, E, X_q, E_q, edge_index_g, graph_id_g, edge_index_q,
           graph_id_q, W_g_emb, b_g_emb, W_q_emb, b_q_emb, beta_g, beta_q,
           red_W1, red_b1, red_W2, red_b2, Wp1, bp1, Wp2, bp2):
    n_g = X.shape[0]
    n_q = X_q.shape[0]
    src_g = edge_index_g[0]
    dst_g = edge_index_g[1]
    src_q = edge_index_q[0]
    dst_q = edge_index_q[1]
    gid3_g = graph_id_g.reshape(n_g // 1000, 1, 1000)
    gid3_q = graph_id_q.reshape(n_q // 1000, 1, 1000)
    zgram = jnp.zeros((NUM_GRAPHS * NUM_GRAPHS,), jnp.float32)

    h_g = _mm_bias(X, W_g_emb, b_g_emb, 1000)
    h_q = _mm_bias(X_q, W_q_emb, b_q_emb, 1000)

    for l in range(L):
        bq16 = jnp.broadcast_to(beta_q[l], (16,)).astype(jnp.float32)
        bg16 = jnp.broadcast_to(beta_g[l], (16,)).astype(jnp.float32)

        hn_q, rinv_q, t0q, t1q = _prep_q(h_q, gid3_q, 1000)
        if _DBG_XLA_DOT:
            w_q = _xla_edge_dot(hn_q, rinv_q.reshape(-1), graph_id_q,
                                zgram, src_q, dst_q, bq16)
        else:
            w_q = _edge_dot_q(hn_q, rinv_q.reshape(-1), graph_id_q, zgram,
                              src_q, dst_q, bq16)
        if _DBG_XLA_AGGR:
            oq0, oq1 = _xla_aggr(t0q, t1q, src_q, dst_q, w_q, n_q)
        else:
            oq0, oq1 = _aggr_q(t0q, t1q, src_q, dst_q, w_q)
        h_q, aggr_q = _post_q(oq0, oq1, gid3_q, 128, 1000)

        hn_g, rinv_g, t0g, t1g, gram = _prep_g(h_g, aggr_q, gid3_g, 1000)
        if _DBG_XLA_DOT:
            w_g = _xla_edge_dot(hn_g, rinv_g.reshape(-1), graph_id_g,
                                gram.reshape(-1), src_g, dst_g, bg16)
        else:
            w_g = _edge_dot_g(hn_g, rinv_g.reshape(-1), graph_id_g,
                              gram.reshape(-1), src_g, dst_g, bg16)
        if _DBG_XLA_AGGR:
            og0, og1 = _xla_aggr(t0g, t1g, src_g, dst_g, w_g, n_g)
        else:
            og0, og1 = _aggr_g(t0g, t1g, src_g, dst_g, w_g)
        h_g = _post_g(og0, og1, aggr_q, red_W1[l], red_b1[l], red_W2[l],
                      red_b2[l], 128, 1000)

    hg_sum = _segsum(h_g, gid3_g, 1000)
    hq_sum = _segsum(h_q, gid3_q, 1000)
    y = _head(hg_sum, hq_sum, Wp1, bp1, Wp2, bp2)
    if _DIAG is not None:
        y = y + jnp.pad(_DIAG[None, None], ((0, NUM_GRAPHS - 1), (0, 0)))
    return y


# speedup vs baseline: 2.9226x; 1.0002x over previous
import functools

import jax
import jax.numpy as jnp
from jax import lax
from jax.experimental import pallas as pl
from jax.experimental.pallas import tpu as pltpu
from jax.experimental.pallas import tpu_sc as plsc

L = 2
NUM_GRAPHS = 16
HALF = 64
TW = 128


def _make_edge_dot(n_nodes, n_edges, chunk):
    d_feat = 128
    per_tile = n_edges // 32
    n_chunks = per_tile // chunk
    groups = -(-chunk // 16)
    cpad = groups * 16
    mesh = plsc.VectorSubcoreMesh(core_axis_name="c", subcore_axis_name="s")

    @functools.partial(
        pl.kernel, mesh=mesh, name=f"edge_dot_{n_edges}",
        out_type=jax.ShapeDtypeStruct((n_edges,), jnp.float32),
        compiler_params=pltpu.CompilerParams(needs_layout_passes=False),
        scratch_types=[
            pltpu.VMEM((cpad,), jnp.int32),
            pltpu.VMEM((cpad,), jnp.int32),
            pltpu.VMEM((cpad, d_feat), jnp.float32),
            pltpu.VMEM((cpad, d_feat), jnp.float32),
            pltpu.VMEM((cpad,), jnp.float32),
            pltpu.VMEM((16,), jnp.float32),
            pltpu.VMEM((n_nodes,), jnp.int32),
            pltpu.VMEM((n_nodes,), jnp.float32),
            pltpu.VMEM((NUM_GRAPHS * NUM_GRAPHS,), jnp.float32),
            pltpu.SemaphoreType.DMA,
            pltpu.SemaphoreType.DMA,
        ],
    )
    def k(hn_hbm, rinv_hbm, gid_hbm, gram_hbm, src_hbm, dst_hbm, beta_hbm,
          w_hbm, si, di, S, T, wb, bb, gid_v, rinv_v, gram_v, sem1, sem2):
        cid = lax.axis_index("c")
        sid = lax.axis_index("s")
        wid = sid * 2 + cid
        base0 = wid * per_tile
        pltpu.sync_copy(beta_hbm, bb)
        pltpu.sync_copy(gid_hbm, gid_v)
        pltpu.sync_copy(rinv_hbm, rinv_v)
        pltpu.sync_copy(gram_hbm, gram_v)
        bv = bb[...]
        lanes = lax.iota(jnp.int32, 16)
        if cpad != chunk:
            si[pl.ds(cpad - 16, 16)] = jnp.zeros((16,), jnp.int32)
            di[pl.ds(cpad - 16, 16)] = jnp.zeros((16,), jnp.int32)

        def chunk_body(i, carry):
            base = base0 + i * chunk
            pltpu.sync_copy(src_hbm.at[pl.ds(base, chunk)],
                            si.at[pl.ds(0, chunk)])
            pltpu.sync_copy(dst_hbm.at[pl.ds(base, chunk)],
                            di.at[pl.ds(0, chunk)])
            cp1 = pltpu.async_copy(hn_hbm.at[si], S, sem1)
            cp2 = pltpu.async_copy(hn_hbm.at[di], T, sem2)
            cp1.wait()
            cp2.wait()
            for g in range(groups):
                rows = g * 16 + lanes
                s16 = si[pl.ds(g * 16, 16)]
                d16 = di[pl.ds(g * 16, 16)]
                gs = plsc.load_gather(gid_v, [s16])
                gd = plsc.load_gather(gid_v, [d16])
                rs = plsc.load_gather(rinv_v, [s16])
                rd = plsc.load_gather(rinv_v, [d16])
                gv = plsc.load_gather(gram_v, [gs * NUM_GRAPHS + gd])

                def dot_body(j, acc):
                    col0 = j * 8
                    for kk in range(8):
                        cv = jnp.full((16,), col0 + kk, jnp.int32)
                        a = plsc.load_gather(S, [rows, cv])
                        b = plsc.load_gather(T, [rows, cv])
                        acc = acc + a * b
                    return acc

                acc = lax.fori_loop(0, d_feat // 8, dot_body,
                                    jnp.zeros((16,), jnp.float32))
                e = acc + gv * rs * rd
                wb[pl.ds(g * 16, 16)] = jnp.exp(e * bv)
            pltpu.sync_copy(wb.at[pl.ds(0, chunk)],
                            w_hbm.at[pl.ds(base, chunk)])
            return carry

        lax.fori_loop(0, n_chunks, chunk_body, 0)

    return k


def _make_aggregate(n_nodes, n_edges, chunk):
    per_tile = n_edges // 16
    n_chunks = per_tile // chunk
    stripe = -(-(n_nodes // 16) // 8) * 8
    last_rows = n_nodes - 15 * stripe
    assert last_rows > 0 and last_rows % 8 == 0
    mrows = 120
    full_a, tail_a = divmod(stripe, mrows)
    full_b, tail_b = divmod(last_rows, mrows)
    kgroups = TW // 16
    mesh = plsc.VectorSubcoreMesh(core_axis_name="c", subcore_axis_name="s")
    osd = jax.ShapeDtypeStruct((n_nodes, TW), jnp.float32)

    @functools.partial(
        pl.kernel, mesh=mesh, name=f"aggr_{n_edges}",
        out_type=(osd, osd),
        compiler_params=pltpu.CompilerParams(needs_layout_passes=False),
        scratch_types=[
            pltpu.VMEM((chunk,), jnp.int32),
            pltpu.VMEM((chunk,), jnp.int32),
            pltpu.VMEM((chunk,), jnp.float32),
            pltpu.VMEM((chunk, TW), jnp.float32),
            pltpu.VMEM((mrows, TW), jnp.float32),
            pltpu.VMEM_SHARED((n_nodes, TW), jnp.float32),
            pltpu.SemaphoreType.DMA,
        ],
    )
    def k(t0_hbm, t1_hbm, src_hbm, dst_hbm, w_hbm, o0_hbm, o1_hbm,
          si, di, wv, R, Z, acc, sem):
        cid = lax.axis_index("c")
        sid = lax.axis_index("s")

        def zrow(r, carry):
            for kk in range(kgroups):
                Z[r, pl.ds(kk * 16, 16)] = jnp.zeros((16,), jnp.float32)
            return carry

        lax.fori_loop(0, mrows, zrow, 0)

        def stripe_copy(fn):
            base = sid * stripe

            def full_chunks(nfull, tail):
                def cb(m, carry):
                    fn(base + m * mrows, mrows)
                    return carry
                lax.fori_loop(0, nfull, cb, 0)
                if tail:
                    fn(base + nfull * mrows, tail)

            if (full_a, tail_a) == (full_b, tail_b):
                full_chunks(full_a, tail_a)
            else:
                pl.when(sid != 15)(lambda: full_chunks(full_a, tail_a))
                pl.when(sid == 15)(lambda: full_chunks(full_b, tail_b))

        stripe_copy(lambda r0, nr: pltpu.sync_copy(
            Z.at[pl.ds(0, nr)], acc.at[pl.ds(r0, nr)]))
        plsc.subcore_barrier()

        def run(t_hbm):
            base0 = sid * per_tile

            def cb(i, carry):
                base = base0 + i * chunk
                pltpu.sync_copy(src_hbm.at[pl.ds(base, chunk)], si)
                pltpu.sync_copy(dst_hbm.at[pl.ds(base, chunk)], di)
                pltpu.sync_copy(w_hbm.at[pl.ds(base, chunk)], wv)
                pltpu.async_copy(t_hbm.at[si], R, sem).wait()

                def erow(e, carry2):
                    ws = plsc.load_gather(
                        wv, [jnp.full((16,), e, jnp.int32)])
                    for kk in range(kgroups):
                        R[e, pl.ds(kk * 16, 16)] = (
                            R[e, pl.ds(kk * 16, 16)] * ws)
                    return carry2

                lax.fori_loop(0, chunk, erow, 0)
                pltpu.sync_copy(R, acc.at[di], add=True)
                return carry

            lax.fori_loop(0, n_chunks, cb, 0)

        pl.when(cid == 0)(lambda: run(t0_hbm))
        pl.when(cid == 1)(lambda: run(t1_hbm))
        plsc.subcore_barrier()

        def dump(o_hbm):
            def one(r0, nr):
                pltpu.sync_copy(acc.at[pl.ds(r0, nr)], Z.at[pl.ds(0, nr)])
                pltpu.sync_copy(Z.at[pl.ds(0, nr)], o_hbm.at[pl.ds(r0, nr)])
            stripe_copy(one)

        pl.when(cid == 0)(lambda: dump(o0_hbm))
        pl.when(cid == 1)(lambda: dump(o1_hbm))

    return k


def _mm_bias_body(x_ref, w_ref, b_ref, o_ref):
    o_ref[...] = (
        jnp.dot(x_ref[...], w_ref[...], preferred_element_type=jnp.float32)
        + b_ref[...]
    )


def _mm_bias(x, w, b, block_rows):
    n, kdim = x.shape
    m = w.shape[1]
    return pl.pallas_call(
        _mm_bias_body,
        grid=(n // block_rows,),
        in_specs=[
            pl.BlockSpec((block_rows, kdim), lambda i: (i, 0)),
            pl.BlockSpec((kdim, m), lambda i: (0, 0)),
            pl.BlockSpec((m,), lambda i: (0,)),
        ],
        out_specs=pl.BlockSpec((block_rows, m), lambda i: (i, 0)),
        out_shape=jax.ShapeDtypeStruct((n, m), jnp.float32),
    )(x, w, b)


def _onehot(gid):
    return (gid[:, None] == lax.iota(jnp.int32, NUM_GRAPHS)[None, :]
            ).astype(jnp.float32)


def _tables(h, oh, rinv):
    r = h.shape[0]
    hn = h * rinv[:, None]
    zer = jnp.zeros((r, TW - HALF - NUM_GRAPHS), jnp.float32)
    t0 = jnp.concatenate([h[:, :HALF], oh, zer], axis=1)
    t1 = jnp.concatenate([h[:, HALF:], oh, zer], axis=1)
    return hn, t0, t1


def _prep_q_body(h_ref, gid_ref, hn_ref, rinv_ref, t0_ref, t1_ref):
    h = h_ref[...]
    rinv = 1.0 / (jnp.sqrt(jnp.sum(h * h, axis=1)) + 1e-12)
    oh = _onehot(gid_ref[0, 0, :])
    hn, t0, t1 = _tables(h, oh, rinv)
    hn_ref[...] = hn
    rinv_ref[...] = rinv[:, None]
    t0_ref[...] = t0
    t1_ref[...] = t1


def _prep_q(h, gid3, block_rows):
    n, d = h.shape
    return pl.pallas_call(
        _prep_q_body,
        grid=(n // block_rows,),
        in_specs=[
            pl.BlockSpec((block_rows, d), lambda i: (i, 0)),
            pl.BlockSpec((1, 1, block_rows), lambda i: (i, 0, 0)),
        ],
        out_specs=[
            pl.BlockSpec((block_rows, d), lambda i: (i, 0)),
            pl.BlockSpec((block_rows, 1), lambda i: (i, 0)),
            pl.BlockSpec((block_rows, TW), lambda i: (i, 0)),
            pl.BlockSpec((block_rows, TW), lambda i: (i, 0)),
        ],
        out_shape=[
            jax.ShapeDtypeStruct((n, d), jnp.float32),
            jax.ShapeDtypeStruct((n, 1), jnp.float32),
            jax.ShapeDtypeStruct((n, TW), jnp.float32),
            jax.ShapeDtypeStruct((n, TW), jnp.float32),
        ],
    )(h, gid3)


def _prep_g_body(h_ref, aggr_ref, gid_ref, hn_ref, rinv_ref, t0_ref,
                 t1_ref, gram_ref):
    h = h_ref[...]
    aggr = aggr_ref[...]
    oh = _onehot(gid_ref[0, 0, :])
    nag = jnp.sum(aggr * aggr, axis=1)
    n2 = jnp.sum(h * h, axis=1) + jnp.dot(
        oh, nag[:, None], preferred_element_type=jnp.float32)[:, 0]
    rinv = 1.0 / (jnp.sqrt(n2) + 1e-12)
    hn, t0, t1 = _tables(h, oh, rinv)
    hn_ref[...] = hn
    rinv_ref[...] = rinv[:, None]
    t0_ref[...] = t0
    t1_ref[...] = t1
    gram_ref[...] = lax.dot_general(
        aggr, aggr, (((1,), (1,)), ((), ())),
        preferred_element_type=jnp.float32)


def _prep_g(h, aggr, gid3, block_rows):
    n, d = h.shape
    return pl.pallas_call(
        _prep_g_body,
        grid=(n // block_rows,),
        in_specs=[
            pl.BlockSpec((block_rows, d), lambda i: (i, 0)),
            pl.BlockSpec((NUM_GRAPHS, d), lambda i: (0, 0)),
            pl.BlockSpec((1, 1, block_rows), lambda i: (i, 0, 0)),
        ],
        out_specs=[
            pl.BlockSpec((block_rows, d), lambda i: (i, 0)),
            pl.BlockSpec((block_rows, 1), lambda i: (i, 0)),
            pl.BlockSpec((block_rows, TW), lambda i: (i, 0)),
            pl.BlockSpec((block_rows, TW), lambda i: (i, 0)),
            pl.BlockSpec((NUM_GRAPHS, NUM_GRAPHS), lambda i: (0, 0)),
        ],
        out_shape=[
            jax.ShapeDtypeStruct((n, d), jnp.float32),
            jax.ShapeDtypeStruct((n, 1), jnp.float32),
            jax.ShapeDtypeStruct((n, TW), jnp.float32),
            jax.ShapeDtypeStruct((n, TW), jnp.float32),
            jax.ShapeDtypeStruct((NUM_GRAPHS, NUM_GRAPHS), jnp.float32),
        ],
    )(h, aggr, gid3)


def _post_q_body(o0_ref, o1_ref, gid_ref, hq_ref, aggr_ref):
    i = pl.program_id(0)
    o0 = o0_ref[...]
    o1 = o1_ref[...]
    den = jnp.sum(o0[:, HALF:HALF + NUM_GRAPHS], axis=1)
    r = (1.0 / (den + 1e-12))[:, None]
    hq = jnp.concatenate([o0[:, :HALF] * r, o1[:, :HALF] * r], axis=1)
    hq_ref[...] = hq
    oh = _onehot(gid_ref[0, 0, :])
    part = lax.dot_general(oh, hq, (((0,), (0,)), ((), ())),
                           preferred_element_type=jnp.float32)

    @pl.when(i == 0)
    def _():
        aggr_ref[...] = jnp.zeros_like(aggr_ref)

    aggr_ref[...] += part


def _post_q(o0, o1, gid3, d, block_rows):
    n = o0.shape[0]
    return pl.pallas_call(
        _post_q_body,
        grid=(n // block_rows,),
        in_specs=[
            pl.BlockSpec((block_rows, TW), lambda i: (i, 0)),
            pl.BlockSpec((block_rows, TW), lambda i: (i, 0)),
            pl.BlockSpec((1, 1, block_rows), lambda i: (i, 0, 0)),
        ],
        out_specs=[
            pl.BlockSpec((block_rows, d), lambda i: (i, 0)),
            pl.BlockSpec((NUM_GRAPHS, d), lambda i: (0, 0)),
        ],
        out_shape=[
            jax.ShapeDtypeStruct((n, d), jnp.float32),
            jax.ShapeDtypeStruct((NUM_GRAPHS, d), jnp.float32),
        ],
    )(o0, o1, gid3)


def _post_g_body(o0_ref, o1_ref, aggr_ref, w1_ref, b1_ref, w2_ref, b2_ref,
                 hg_ref):
    o0 = o0_ref[...]
    o1 = o1_ref[...]
    cnt = o0[:, HALF:HALF + NUM_GRAPHS]
    den = jnp.sum(cnt, axis=1)
    r = (1.0 / (den + 1e-12))[:, None]
    a = o0[:, :HALF] * r
    b = o1[:, :HALF] * r
    c = jnp.dot(cnt, aggr_ref[...],
                preferred_element_type=jnp.float32) * r
    w1 = w1_ref[...]
    t = jax.nn.relu(
        jnp.dot(a, w1[:HALF], preferred_element_type=jnp.float32)
        + jnp.dot(b, w1[HALF:2 * HALF], preferred_element_type=jnp.float32)
        + jnp.dot(c, w1[2 * HALF:], preferred_element_type=jnp.float32)
        + b1_ref[...])
    hg_ref[...] = (
        jnp.dot(t, w2_ref[...], preferred_element_type=jnp.float32)
        + b2_ref[...])


def _post_g(o0, o1, aggr, w1, b1, w2, b2, d, block_rows):
    n = o0.shape[0]
    return pl.pallas_call(
        _post_g_body,
        grid=(n // block_rows,),
        in_specs=[
            pl.BlockSpec((block_rows, TW), lambda i: (i, 0)),
            pl.BlockSpec((block_rows, TW), lambda i: (i, 0)),
            pl.BlockSpec((NUM_GRAPHS, d), lambda i: (0, 0)),
            pl.BlockSpec((2 * d, d), lambda i: (0, 0)),
            pl.BlockSpec((d,), lambda i: (0,)),
            pl.BlockSpec((d, d), lambda i: (0, 0)),
            pl.BlockSpec((d,), lambda i: (0,)),
        ],
        out_specs=pl.BlockSpec((block_rows, d), lambda i: (i, 0)),
        out_shape=jax.ShapeDtypeStruct((n, d), jnp.float32),
    )(o0, o1, aggr, w1, b1, w2, b2)


def _segsum_body(h_ref, gid_ref, out_ref):
    i = pl.program_id(0)
    oh = _onehot(gid_ref[0, 0, :])
    part = lax.dot_general(oh, h_ref[...], (((0,), (0,)), ((), ())),
                           preferred_element_type=jnp.float32)

    @pl.when(i == 0)
    def _():
        out_ref[...] = jnp.zeros_like(out_ref)

    out_ref[...] += part


def _segsum(h, gid3, block_rows):
    n, d = h.shape
    return pl.pallas_call(
        _segsum_body,
        grid=(n // block_rows,),
        in_specs=[
            pl.BlockSpec((block_rows, d), lambda i: (i, 0)),
            pl.BlockSpec((1, 1, block_rows), lambda i: (i, 0, 0)),
        ],
        out_specs=pl.BlockSpec((NUM_GRAPHS, d), lambda i: (0, 0)),
        out_shape=jax.ShapeDtypeStruct((NUM_GRAPHS, d), jnp.float32),
    )(h, gid3)


def _head_body(a_ref, b_ref, w1_ref, b1_ref, w2_ref, b2_ref, y_ref):
    x = jnp.concatenate([a_ref[...], b_ref[...]], axis=1)
    t = jax.nn.relu(
        jnp.dot(x, w1_ref[...], preferred_element_type=jnp.float32)
        + b1_ref[...])
    y_ref[...] = (
        jnp.dot(t, w2_ref[...], preferred_element_type=jnp.float32)
        + b2_ref[...])


def _head(a, b, w1, b1, w2, b2):
    out_d = w2.shape[1]
    return pl.pallas_call(
        _head_body,
        out_shape=jax.ShapeDtypeStruct((NUM_GRAPHS, out_d), jnp.float32),
    )(a, b, w1, b1, w2, b2)


_DBG_XLA_DOT = False
_DBG_XLA_AGGR = False
_DIAG = None


def _xla_edge_dot(hn, rinv, gid, gram, src, dst, b16):
    e = jnp.sum(hn[src] * hn[dst], axis=1)
    g = gram[gid[src] * NUM_GRAPHS + gid[dst]] * rinv[src] * rinv[dst]
    return jnp.exp((e + g) * b16[0])


def _xla_aggr(t0, t1, src, dst, w, n):
    o0 = jax.ops.segment_sum(w[:, None] * t0[src], dst, num_segments=n)
    o1 = jax.ops.segment_sum(w[:, None] * t1[src], dst, num_segments=n)
    return o0, o1

_edge_dot_g = _make_edge_dot(10000, 320000, 80)
_edge_dot_q = _make_edge_dot(2000, 32000, 40)
_aggr_g = _make_aggregate(10000, 320000, 80)
_aggr_q = _make_aggregate(2000, 32000, 80)


def kernel(X, E, X_q, E_q, edge_index_g, graph_id_g, edge_index_q,
           graph_id_q, W_g_emb, b_g_emb, W_q_emb, b_q_emb, beta_g, beta_q,
           red_W1, red_b1, red_W2, red_b2, Wp1, bp1, Wp2, bp2):
    n_g = X.shape[0]
    n_q = X_q.shape[0]
    src_g = edge_index_g[0]
    dst_g = edge_index_g[1]
    src_q = edge_index_q[0]
    dst_q = edge_index_q[1]
    gid3_g = graph_id_g.reshape(n_g // 1000, 1, 1000)
    gid3_q = graph_id_q.reshape(n_q // 1000, 1, 1000)
    zgram = jnp.zeros((NUM_GRAPHS * NUM_GRAPHS,), jnp.float32)

    h_g = _mm_bias(X, W_g_emb, b_g_emb, 1000)
    h_q = _mm_bias(X_q, W_q_emb, b_q_emb, 1000)

    for l in range(L):
        bq16 = jnp.broadcast_to(beta_q[l], (16,)).astype(jnp.float32)
        bg16 = jnp.broadcast_to(beta_g[l], (16,)).astype(jnp.float32)

        hn_q, rinv_q, t0q, t1q = _prep_q(h_q, gid3_q, 1000)
        if _DBG_XLA_DOT:
            w_q = _xla_edge_dot(hn_q, rinv_q.reshape(-1), graph_id_q,
                                zgram, src_q, dst_q, bq16)
        else:
            w_q = _edge_dot_q(hn_q, rinv_q.reshape(-1), graph_id_q, zgram,
                              src_q, dst_q, bq16)
        if _DBG_XLA_AGGR:
            oq0, oq1 = _xla_aggr(t0q, t1q, src_q, dst_q, w_q, n_q)
        else:
            oq0, oq1 = _aggr_q(t0q, t1q, src_q, dst_q, w_q)
        h_q, aggr_q = _post_q(oq0, oq1, gid3_q, 128, 1000)

        hn_g, rinv_g, t0g, t1g, gram = _prep_g(h_g, aggr_q, gid3_g, 1000)
        if _DBG_XLA_DOT:
            w_g = _xla_edge_dot(hn_g, rinv_g.reshape(-1), graph_id_g,
                                gram.reshape(-1), src_g, dst_g, bg16)
        else:
            w_g = _edge_dot_g(hn_g, rinv_g.reshape(-1), graph_id_g,
                              gram.reshape(-1), src_g, dst_g, bg16)
        if _DBG_XLA_AGGR:
            og0, og1 = _xla_aggr(t0g, t1g, src_g, dst_g, w_g, n_g)
        else:
            og0, og1 = _aggr_g(t0g, t1g, src_g, dst_g, w_g)
        h_g = _post_g(og0, og1, aggr_q, red_W1[l], red_b1[l], red_W2[l],
                      red_b2[l], 128, 1000)

    hg_sum = _segsum(h_g, gid3_g, 1000)
    hq_sum = _segsum(h_q, gid3_q, 1000)
    y = _head(hg_sum, hq_sum, Wp1, bp1, Wp2, bp2)
    if _DIAG is not None:
        y = y + jnp.pad(_DIAG[None, None], ((0, NUM_GRAPHS - 1), (0, 0)))
    return y
